# baseline (speedup 1.0000x reference)
import functools

import jax
import jax.numpy as jnp
from jax import lax
from jax.experimental import pallas as pl
from jax.experimental.pallas import tpu as pltpu
from jax.experimental.pallas import tpu_sc as plsc

_B = 16384
_D = 32
_FF = 13
_NF = 26
_NR = _NF + 1
_NCOLS = _FF + _NF
_VOCAB = 100000

_NC = 2
_NS = 16
_NW = _NC * _NS
_BW = _B // _NW
_CB = 128
_NSUB = _BW // _CB


def _precompute(features, W, b):
    BS = 2048

    def body(f_ref, w_ref, b_ref, d_ref, i_ref):
        x = f_ref[:, :_FF].astype(jnp.float32)
        d_ref[...] = (
            lax.dot_general(
                x, w_ref[...], (((1,), (1,)), ((), ())),
                preferred_element_type=jnp.float32,
            )
            + b_ref[...]
        )
        f26 = lax.broadcasted_iota(jnp.int32, (BS, _NF), 1)
        i_ref[...] = f_ref[:, _FF:] + f26 * _VOCAB

    return pl.pallas_call(
        body,
        grid=(_B // BS,),
        in_specs=[
            pl.BlockSpec((BS, _NCOLS), lambda i: (i, 0)),
            pl.BlockSpec((_D, _FF), lambda i: (0, 0)),
            pl.BlockSpec((1, _D), lambda i: (0, 0)),
        ],
        out_specs=[
            pl.BlockSpec((BS, _D), lambda i: (i, 0)),
            pl.BlockSpec((BS, _NF), lambda i: (i, 0)),
        ],
        out_shape=[
            jax.ShapeDtypeStruct((_B, _D), jnp.float32),
            jax.ShapeDtypeStruct((_B, _NF), jnp.int32),
        ],
    )(features, W, b.reshape(1, _D))


def _sc_assemble(idx, dense, table):
    mesh = plsc.VectorSubcoreMesh(core_axis_name="c", subcore_axis_name="s")

    @functools.partial(
        pl.kernel,
        mesh=mesh,
        compiler_params=pltpu.CompilerParams(use_tc_tiling_on_sc=False),
        out_type=jax.ShapeDtypeStruct((_B * _NR, _D), jnp.float32),
        scratch_types=[
            pltpu.VMEM((_CB, _NF), jnp.int32),
            pltpu.VMEM((_CB * _NR, _D), jnp.float32),
            pltpu.SemaphoreType.DMA,
            pltpu.SemaphoreType.DMA,
        ],
    )
    def k(idx_hbm, dense_hbm, table_hbm, out_hbm, idx_v, gbuf, gsem, dsem):
        wid = lax.axis_index("s") * _NC + lax.axis_index("c")
        base = wid * _BW

        for sub in range(_NSUB):
            b0 = base + sub * _CB
            pltpu.sync_copy(idx_hbm.at[pl.ds(b0, _CB)], idx_v)

            def fire(bb, carry):
                pltpu.async_copy(
                    table_hbm.at[idx_v.at[bb]],
                    gbuf.at[pl.ds(bb * _NR + 1, _NF)],
                    gsem,
                )
                pltpu.async_copy(
                    dense_hbm.at[pl.ds(b0 + bb, 1)],
                    gbuf.at[pl.ds(bb * _NR, 1)],
                    dsem,
                )
                return carry

            lax.fori_loop(0, _CB, fire, 0)

            def drain(bb, carry):
                pltpu.make_async_copy(
                    table_hbm.at[idx_v.at[0]],
                    gbuf.at[pl.ds(1, _NF)],
                    gsem,
                ).wait()
                pltpu.make_async_copy(
                    dense_hbm.at[pl.ds(b0, 1)],
                    gbuf.at[pl.ds(0, 1)],
                    dsem,
                ).wait()
                return carry

            lax.fori_loop(0, _CB, drain, 0)

            pltpu.sync_copy(gbuf, out_hbm.at[pl.ds(b0 * _NR, _CB * _NR)])

    return k(idx, dense, table)


def kernel(original_features, table, W, b):
    dense, idx = _precompute(original_features, W, b)
    out2d = _sc_assemble(idx, dense, table)
    return out2d.reshape(_B, _NR, _D)


# speedup vs baseline: 1.2359x; 1.2359x over previous
import functools

import jax
import jax.numpy as jnp
from jax import lax
from jax.experimental import pallas as pl
from jax.experimental.pallas import tpu as pltpu
from jax.experimental.pallas import tpu_sc as plsc

_B = 16384
_D = 32
_FF = 13
_NF = 26
_NR = _NF + 1
_NCOLS = _FF + _NF
_VOCAB = 100000

_NC = 2
_NS = 16
_NW = _NC * _NS
_BW = _B // _NW
_CB = 32
_NSUB = _BW // _CB


def _precompute(features, W, b):
    BS = 2048

    def body(f_ref, w_ref, b_ref, d_ref, i_ref):
        x = f_ref[:, :_FF].astype(jnp.float32)
        d_ref[...] = (
            lax.dot_general(
                x, w_ref[...], (((1,), (1,)), ((), ())),
                preferred_element_type=jnp.float32,
            )
            + b_ref[...]
        )
        f26 = lax.broadcasted_iota(jnp.int32, (BS, _NF), 1)
        toks = f_ref[:, _FF:] + f26 * _VOCAB
        i_ref[...] = jnp.concatenate(
            [toks, jnp.zeros((BS, _D - _NF), jnp.int32)], axis=1
        )

    return pl.pallas_call(
        body,
        grid=(_B // BS,),
        in_specs=[
            pl.BlockSpec((BS, _NCOLS), lambda i: (i, 0)),
            pl.BlockSpec((_D, _FF), lambda i: (0, 0)),
            pl.BlockSpec((1, _D), lambda i: (0, 0)),
        ],
        out_specs=[
            pl.BlockSpec((BS, _D), lambda i: (i, 0)),
            pl.BlockSpec((BS, _D), lambda i: (i, 0)),
        ],
        out_shape=[
            jax.ShapeDtypeStruct((_B, _D), jnp.float32),
            jax.ShapeDtypeStruct((_B, _D), jnp.int32),
        ],
    )(features, W, b.reshape(1, _D))


def _sc_assemble(idx, dense, table):
    mesh = plsc.VectorSubcoreMesh(core_axis_name="c", subcore_axis_name="s")

    @functools.partial(
        pl.kernel,
        mesh=mesh,
        out_type=jax.ShapeDtypeStruct((_B * _NR, _D), jnp.float32),
        scratch_types=[
            pltpu.VMEM((_CB, _D), jnp.int32),
            pltpu.VMEM((_CB * _NR, _D), jnp.float32),
            pltpu.SemaphoreType.DMA,
            pltpu.SemaphoreType.DMA,
        ],
    )
    def k(idx_hbm, dense_hbm, table_hbm, out_hbm, idx_v, gbuf, gsem, dsem):
        wid = lax.axis_index("s") * _NC + lax.axis_index("c")
        base = wid * _BW

        def sub_body(sub, carry):
            b0 = base + sub * _CB
            pltpu.sync_copy(idx_hbm.at[pl.ds(b0, _CB)], idx_v)

            def fire(bb, c2):
                pltpu.async_copy(
                    dense_hbm.at[pl.ds(b0 + bb, 1)],
                    gbuf.at[pl.ds(bb * _NR, 1)],
                    dsem,
                )
                t0 = idx_v[bb, pl.ds(0, 16)]
                t1 = idx_v[bb, pl.ds(16, 16)]
                for f in range(_NF):
                    tok = t0[f] if f < 16 else t1[f - 16]
                    pltpu.async_copy(
                        table_hbm.at[pl.ds(tok, 1)],
                        gbuf.at[pl.ds(bb * _NR + 1 + f, 1)],
                        gsem,
                    )
                return c2

            lax.fori_loop(0, _CB, fire, 0)

            def drain(bb, c2):
                pltpu.make_async_copy(
                    dense_hbm.at[pl.ds(0, 1)],
                    gbuf.at[pl.ds(0, 1)],
                    dsem,
                ).wait()
                for f in range(_NF):
                    pltpu.make_async_copy(
                        table_hbm.at[pl.ds(0, 1)],
                        gbuf.at[pl.ds(0, 1)],
                        gsem,
                    ).wait()
                return c2

            lax.fori_loop(0, _CB, drain, 0)

            pltpu.sync_copy(gbuf, out_hbm.at[pl.ds(b0 * _NR, _CB * _NR)])
            return carry

        lax.fori_loop(0, _NSUB, sub_body, 0)

    return k(idx, dense, table)


def kernel(original_features, table, W, b):
    dense, idx = _precompute(original_features, W, b)
    out2d = _sc_assemble(idx, dense, table)
    return out2d.reshape(_B, _NR, _D)
